# Initial kernel scaffold; baseline (speedup 1.0000x reference)
import jax
import jax.numpy as jnp
from jax.experimental import pallas as pl


def kernel(x, table):
    raise NotImplementedError("write your pallas kernel here")


# speedup vs baseline: 8.2056x; 8.2056x over previous
import functools

import jax
import jax.numpy as jnp
from jax import lax
from jax.experimental import pallas as pl
from jax.experimental.pallas import tpu as pltpu
from jax.experimental.pallas import tpu_sc as plsc

VOCAB = 100000
D = 128
NUM_CORES = 2
NUM_SUBCORES = 16
NW = NUM_CORES * NUM_SUBCORES
IDX_W = 128


def _make_gather(n_rows_total):
    rows_per_w = n_rows_total // NW
    K = 4
    n_steps = rows_per_w // K
    mesh = plsc.VectorSubcoreMesh(core_axis_name="c", subcore_axis_name="s")

    @functools.partial(
        pl.kernel,
        mesh=mesh,
        out_type=jax.ShapeDtypeStruct((n_rows_total * IDX_W, D), jnp.float32),
        scratch_types=[
            pltpu.VMEM((K, IDX_W), jnp.int32),
            pltpu.VMEM((K * IDX_W, D), jnp.float32),
            pltpu.SemaphoreType.DMA,
        ],
    )
    def gather_kernel(idx_hbm, table_hbm, out_hbm, idx_v, rows_v, sem):
        wid = lax.axis_index("s") * NUM_CORES + lax.axis_index("c")
        row0 = wid * rows_per_w

        def step(g, carry):
            r = row0 + g * K
            pltpu.sync_copy(idx_hbm.at[pl.ds(r, K)], idx_v)
            copies = [
                pltpu.async_copy(
                    table_hbm.at[idx_v.at[j]],
                    rows_v.at[pl.ds(j * IDX_W, IDX_W)],
                    sem,
                )
                for j in range(K)
            ]
            for c in copies:
                c.wait()
            pltpu.sync_copy(rows_v, out_hbm.at[pl.ds(r * IDX_W, K * IDX_W)])
            return carry

        lax.fori_loop(0, n_steps, step, 0)

    return gather_kernel


def kernel(x, table):
    B, L = x.shape
    n = B * L
    idx = x.reshape(n // IDX_W, IDX_W)
    out = _make_gather(n // IDX_W)(idx, table)
    return out.reshape(B, L, D)


# speedup vs baseline: 9.1966x; 1.1208x over previous
import functools

import jax
import jax.numpy as jnp
from jax import lax
from jax.experimental import pallas as pl
from jax.experimental.pallas import tpu as pltpu
from jax.experimental.pallas import tpu_sc as plsc

D = 128
NUM_CORES = 2
NUM_SUBCORES = 16
NW = NUM_CORES * NUM_SUBCORES
IDX_W = 128
K = 2
NBUF = 2


def _make_gather(n_rows_total):
    rows_per_w = n_rows_total // NW
    n_steps = rows_per_w // K
    mesh = plsc.VectorSubcoreMesh(core_axis_name="c", subcore_axis_name="s")

    @functools.partial(
        pl.kernel,
        mesh=mesh,
        out_type=jax.ShapeDtypeStruct((n_rows_total * IDX_W, D), jnp.float32),
        scratch_types=[
            pltpu.VMEM((rows_per_w, IDX_W), jnp.int32),
            pltpu.VMEM((NBUF, K * IDX_W, D), jnp.float32),
            pltpu.SemaphoreType.DMA,
            pltpu.SemaphoreType.DMA,
            pltpu.SemaphoreType.DMA,
            pltpu.SemaphoreType.DMA,
        ],
    )
    def gather_kernel(idx_hbm, table_hbm, out_hbm, idx_v, rows_v, g0, g1, o0, o1):
        wid = lax.axis_index("s") * NUM_CORES + lax.axis_index("c")
        row0 = wid * rows_per_w
        gsem = [g0, g1]
        osem = [o0, o1]

        pltpu.sync_copy(idx_hbm.at[pl.ds(row0, rows_per_w)], idx_v)

        def fire_gathers(g, b):
            for j in range(K):
                pltpu.async_copy(
                    table_hbm.at[idx_v.at[g * K + j]],
                    rows_v.at[b].at[pl.ds(j * IDX_W, IDX_W)],
                    gsem[b],
                )

        def wait_gathers(b):
            for j in range(K):
                pltpu.make_async_copy(
                    table_hbm.at[idx_v.at[0]],
                    rows_v.at[b].at[pl.ds(j * IDX_W, IDX_W)],
                    gsem[b],
                ).wait()

        def fire_out(g, b):
            pltpu.async_copy(
                rows_v.at[b],
                out_hbm.at[pl.ds((row0 + g * K) * IDX_W, K * IDX_W)],
                osem[b],
            )

        def wait_out(b):
            pltpu.make_async_copy(
                rows_v.at[b],
                out_hbm.at[pl.ds(0, K * IDX_W)],
                osem[b],
            ).wait()

        fire_gathers(0, 0)

        @pl.loop(0, n_steps, step=NBUF)
        def _(g_base):
            for b in range(NBUF):
                g = g_base + b
                nb = (b + 1) % NBUF

                @pl.when(jnp.logical_and(g >= 1, g + 1 < n_steps))
                def _():
                    wait_out(nb)

                @pl.when(g + 1 < n_steps)
                def _():
                    fire_gathers(g + 1, nb)

                wait_gathers(b)
                fire_out(g, b)

        for b in range(NBUF):
            wait_out(b)

    return gather_kernel


def kernel(x, table):
    B, L = x.shape
    n = B * L
    idx = x.reshape(n // IDX_W, IDX_W)
    out = _make_gather(n // IDX_W)(idx, table)
    return out.reshape(B, L, D)
